# Initial kernel scaffold; baseline (speedup 1.0000x reference)
import jax
import jax.numpy as jnp
from jax.experimental import pallas as pl


def kernel(x, edge_index, batch, assignment_index_2, iso_type_2, edge_index_2, batch_2, assignment_index_3, iso_type_3, edge_index_3, batch_3, conv1_Wrel, conv1_Wroot, conv1_b, conv2_Wrel, conv2_Wroot, conv2_b, conv3_Wrel, conv3_Wroot, conv3_b, conv4_Wrel, conv4_Wroot, conv4_b, conv5_Wrel, conv5_Wroot, conv5_b, conv6_Wrel, conv6_Wroot, conv6_b, conv7_Wrel, conv7_Wroot, conv7_b, fc1_W, fc1_b, fc2_W, fc2_b, fc3_W, fc3_b):
    raise NotImplementedError("write your pallas kernel here")


# speedup vs baseline: 1.8679x; 1.8679x over previous
import functools

import jax
import jax.numpy as jnp
from jax import lax
from jax.experimental import pallas as pl
from jax.experimental.pallas import tpu as pltpu
from jax.experimental.pallas import tpu_sc as plsc

NC, NS, LN = 2, 16, 16
KB = 2048
TI = KB // 128
WT = 1000

N1, G = 10000, 256
N2, I2 = 160000, 12
N3, I3 = 80000, 20


def _elu(v):
    return jnp.where(v > 0, v, jnp.exp(v) - 1.0)


def _pad_edges(src, dst, n_out):
    e = src.shape[0]
    step = NS * KB
    pe = ((e + step - 1) // step) * step
    if pe != e:
        src = jnp.concatenate([src, jnp.zeros((pe - e,), src.dtype)])
        dst = jnp.concatenate([dst, jnp.full((pe - e,), n_out, dst.dtype)])
    return src, dst


def _sc_segsum(y_flat, src, dst, *, s_slices, n_out, n_chunks, with_counts):
    PE = src.shape[0]
    NB = PE // (NS * KB)
    CH = n_out // n_chunks
    assert CH % WT == 0
    NT = CH // WT
    QT = -(-NT // NS)
    SL = s_slices // NC
    S = s_slices

    outs = [jax.ShapeDtypeStruct((S, n_out, LN), jnp.float32)]
    if with_counts:
        outs.append(jax.ShapeDtypeStruct((n_out, LN), jnp.float32))
    scratch = [
        pltpu.VMEM((KB,), jnp.int32),
        pltpu.VMEM((KB,), jnp.int32),
        pltpu.VMEM((TI, 128), jnp.int32),
        pltpu.VMEM((TI, 128), jnp.int32),
        pltpu.VMEM((TI, 128, LN), jnp.float32),
        pltpu.VMEM((WT, LN), jnp.float32),
        pltpu.VMEM((WT, LN), jnp.float32),
        pltpu.VMEM_SHARED((CH + 8, LN), jnp.float32),
        pltpu.SemaphoreType.DMA,
    ]
    mesh = plsc.VectorSubcoreMesh(core_axis_name="c", subcore_axis_name="s",
                                  num_cores=NC, num_subcores=NS)

    def body(y_ref, src_ref, dst_ref, *rest):
        if with_counts:
            out_ref, cnt_ref = rest[0], rest[1]
            sc = rest[2:]
        else:
            out_ref = rest[0]
            cnt_ref = None
            sc = rest[1:]
        src_v, dst_v, gidx, didx, rows, zbuf, wbuf, acc, sem = sc
        cid = lax.axis_index("c")
        sid = lax.axis_index("s")
        ebase = sid * (PE // NS)

        z16 = jnp.zeros((LN,), jnp.float32)

        def zfill(i, carry):
            zbuf[i] = z16
            return carry
        lax.fori_loop(0, WT, zfill, 0)

        def zero_acc():
            for q in range(QT):
                t = sid + q * NS
                @pl.when(t < NT)
                def _():
                    pltpu.sync_copy(zbuf, acc.at[pl.ds(t * WT, WT)])

        def run_edges(r0, s, count):
            def bstep(b, carry):
                eb = ebase + b * KB
                pltpu.sync_copy(dst_ref.at[pl.ds(eb, KB)], dst_v)
                if not count:
                    pltpu.sync_copy(src_ref.at[pl.ds(eb, KB)], src_v)

                def istep(j, c2):
                    row = j // 8
                    col = (j % 8) * LN
                    dv = dst_v[pl.ds(j * LN, LN)]
                    dl = dv - r0
                    okm = (dl >= 0) & (dl < CH)
                    didx[row, pl.ds(col, LN)] = jnp.where(okm, dl, CH)
                    if not count:
                        sv = src_v[pl.ds(j * LN, LN)]
                        gidx[row, pl.ds(col, LN)] = sv * S + s
                    return c2
                lax.fori_loop(0, KB // LN, istep, 0)

                if not count:
                    descs = [pltpu.async_copy(y_ref.at[gidx.at[t]],
                                              rows.at[t], sem)
                             for t in range(TI)]
                    for d in descs:
                        d.wait()
                    for t in range(TI):
                        pltpu.sync_copy(rows.at[t], acc.at[didx.at[t]],
                                        add=True)
                else:
                    for t in range(TI):
                        pltpu.sync_copy(rows.at[0], acc.at[didx.at[t]],
                                        add=True)
                return carry
            lax.fori_loop(0, NB, bstep, 0)

        def writeback(emit):
            for q in range(QT):
                t = sid + q * NS
                @pl.when(t < NT)
                def _():
                    r = t * WT
                    pltpu.sync_copy(acc.at[pl.ds(r, WT)], wbuf)
                    pltpu.sync_copy(wbuf, emit(r))

        for sl in range(SL):
            s = cid * SL + sl
            for c in range(n_chunks):
                r0 = c * CH
                zero_acc()
                plsc.subcore_barrier()
                run_edges(r0, s, False)
                plsc.subcore_barrier()
                writeback(lambda r, s=s, r0=r0:
                          out_ref.at[s, pl.ds(r0 + r, WT)])
                plsc.subcore_barrier()

        if with_counts:
            o16 = jnp.ones((LN,), jnp.float32)

            def ofill(i, carry):
                rows[0, i] = o16
                return carry
            lax.fori_loop(0, 128, ofill, 0)
            for q in range(max(1, n_chunks // NC)):
                if n_chunks == 1:
                    r0c = 0
                else:
                    r0c = (q * NC + cid) * CH
                zero_acc()
                plsc.subcore_barrier()
                run_edges(r0c, 0, True)
                plsc.subcore_barrier()
                writeback(lambda r, r0c=r0c: cnt_ref.at[pl.ds(r0c + r, WT)])
                plsc.subcore_barrier()

    f = pl.kernel(body, out_type=tuple(outs) if with_counts else outs[0],
                  mesh=mesh, scratch_types=scratch,
                  compiler_params=pltpu.CompilerParams(
                      use_tc_tiling_on_sc=False))
    return f(y_flat, src, dst)


RB = 1000


def _full(spec_shape):
    return pl.BlockSpec(spec_shape, lambda i: tuple(0 for _ in spec_shape))


def _rows(spec_shape, dim=0):
    def imap(i):
        idx = [0] * len(spec_shape)
        idx[dim] = i
        return tuple(idx)
    return pl.BlockSpec(spec_shape, imap)


def _tc_first(x, wrel, wroot, b):
    n, fin = x.shape
    fout = wrel.shape[1]

    def body(x_ref, wr_ref, wo_ref, b_ref, y_ref, r_ref):
        xb = x_ref[...]
        y_ref[...] = xb @ wr_ref[...]
        r_ref[...] = xb @ wo_ref[...] + b_ref[...]

    return pl.pallas_call(
        body,
        grid=(n // RB,),
        in_specs=[_rows((RB, fin)), _full((fin, fout)), _full((fin, fout)),
                  _full((1, fout))],
        out_specs=[_rows((RB, fout)), _rows((RB, fout))],
        out_shape=[jax.ShapeDtypeStruct((n, fout), jnp.float32)] * 2,
    )(x, wrel, wroot, b.reshape(1, -1))


def _tc_mid(m_sm, r_in, wrel, wroot, b):
    s, n, _ = m_sm.shape
    w_in = s * LN
    fout = wrel.shape[1]

    def body(m_ref, r_ref, wr_ref, wo_ref, b_ref, y_ref, r2_ref):
        m = jnp.concatenate([m_ref[i] for i in range(s)], axis=1)
        h = _elu(m + r_ref[...])
        y_ref[...] = h @ wr_ref[...]
        r2_ref[...] = h @ wo_ref[...] + b_ref[...]

    return pl.pallas_call(
        body,
        grid=(n // RB,),
        in_specs=[_rows((s, RB, LN), dim=1), _rows((RB, w_in)),
                  _full((w_in, fout)), _full((w_in, fout)), _full((1, fout))],
        out_specs=[_rows((RB, fout)), _rows((RB, fout))],
        out_shape=[jax.ShapeDtypeStruct((n, fout), jnp.float32)] * 2,
    )(m_sm, r_in, wrel, wroot, b.reshape(1, -1))


def _tc_last(m_sm, r_in, seg, emit_h):
    s, n, _ = m_sm.shape
    w = s * LN
    ng = n // RB

    def body(m_ref, r_ref, bt_ref, *rest):
        if emit_h:
            h_ref, x_ref, pool_acc, cnt_acc = rest
        else:
            x_ref, pool_acc, cnt_acc = rest
        i = pl.program_id(0)
        m = jnp.concatenate([m_ref[k] for k in range(s)], axis=1)
        h = _elu(m + r_ref[...])
        if emit_h:
            h_ref[...] = h
        oh = (bt_ref[...] ==
              lax.broadcasted_iota(jnp.int32, (RB, G), 1)).astype(jnp.float32)
        pool = lax.dot_general(oh, h, (((0,), (0,)), ((), ())))
        cnt = lax.dot_general(oh, jnp.ones((RB, w), jnp.float32),
                              (((0,), (0,)), ((), ())))

        @pl.when(i == 0)
        def _():
            pool_acc[...] = pool
            cnt_acc[...] = cnt

        @pl.when(i > 0)
        def _():
            pool_acc[...] += pool
            cnt_acc[...] += cnt

        @pl.when(i == ng - 1)
        def _():
            x_ref[...] = pool_acc[...] / jnp.maximum(cnt_acc[...], 1.0)

    out_specs = [_full((G, w))]
    out_shape = [jax.ShapeDtypeStruct((G, w), jnp.float32)]
    if emit_h:
        out_specs = [_rows((RB, w))] + out_specs
        out_shape = [jax.ShapeDtypeStruct((n, w), jnp.float32)] + out_shape
    res = pl.pallas_call(
        body,
        grid=(ng,),
        in_specs=[_rows((s, RB, LN), dim=1), _rows((RB, w)), _rows((RB, 1))],
        out_specs=out_specs,
        out_shape=out_shape,
        scratch_shapes=[pltpu.VMEM((G, w), jnp.float32),
                        pltpu.VMEM((G, w), jnp.float32)],
    )(m_sm, r_in, seg.reshape(-1, 1))
    return res if emit_h else res[0]


def _tc_post_pool(hp_sm, cnt, iso, wrel, wroot, b):
    s, n, _ = hp_sm.shape
    w = s * LN
    fi = iso.shape[1]
    fout = wrel.shape[1]
    wrh, wri = wrel[:w], wrel[w:]
    woh, woi = wroot[:w], wroot[w:]

    def body(hp_ref, cnt_ref, iso_ref, wrh_ref, wri_ref, woh_ref, woi_ref,
             b_ref, y_ref, r_ref):
        hsum = jnp.concatenate([hp_ref[k] for k in range(s)], axis=1)
        c = jnp.maximum(cnt_ref[...][:, 0:1], 1.0)
        hm = hsum / c
        iso_b = iso_ref[...]
        y_ref[...] = hm @ wrh_ref[...] + iso_b @ wri_ref[...]
        r_ref[...] = hm @ woh_ref[...] + iso_b @ woi_ref[...] + b_ref[...]

    return pl.pallas_call(
        body,
        grid=(n // RB,),
        in_specs=[_rows((s, RB, LN), dim=1), _rows((RB, LN)), _rows((RB, fi)),
                  _full((w, fout)), _full((fi, fout)),
                  _full((w, fout)), _full((fi, fout)), _full((1, fout))],
        out_specs=[_rows((RB, fout)), _rows((RB, fout))],
        out_shape=[jax.ShapeDtypeStruct((n, fout), jnp.float32)] * 2,
    )(hp_sm, cnt, iso, wrh, wri, woh, woi, b.reshape(1, -1))


def _tc_head(x1, x2, x3, f1w, f1b, f2w, f2b, f3w, f3b):
    def body(x1_ref, x2_ref, x3_ref, w1_ref, b1_ref, w2_ref, b2_ref,
             w3_ref, b3_ref, o_ref):
        z = jnp.concatenate([x1_ref[...], x2_ref[...], x3_ref[...]], axis=1)
        z = _elu(z @ w1_ref[...] + b1_ref[...])
        z = _elu(z @ w2_ref[...] + b2_ref[...])
        z = z @ w3_ref[...] + b3_ref[...]
        mx = jnp.max(z, axis=1, keepdims=True)
        lse = jnp.log(jnp.sum(jnp.exp(z - mx), axis=1, keepdims=True)) + mx
        o_ref[...] = z - lse

    c = f3w.shape[1]
    return pl.pallas_call(
        body,
        grid=(1,),
        in_specs=[_full((G, 64)), _full((G, 64)), _full((G, 64)),
                  _full((192, 64)), _full((1, 64)),
                  _full((64, 32)), _full((1, 32)),
                  _full((32, c)), _full((1, c))],
        out_specs=_full((G, c)),
        out_shape=jax.ShapeDtypeStruct((G, c), jnp.float32),
    )(x1, x2, x3, f1w, f1b.reshape(1, -1), f2w, f2b.reshape(1, -1),
      f3w, f3b.reshape(1, -1))


def kernel(x, edge_index, batch, assignment_index_2, iso_type_2, edge_index_2,
           batch_2, assignment_index_3, iso_type_3, edge_index_3, batch_3,
           conv1_Wrel, conv1_Wroot, conv1_b, conv2_Wrel, conv2_Wroot, conv2_b,
           conv3_Wrel, conv3_Wroot, conv3_b, conv4_Wrel, conv4_Wroot, conv4_b,
           conv5_Wrel, conv5_Wroot, conv5_b, conv6_Wrel, conv6_Wroot, conv6_b,
           conv7_Wrel, conv7_Wroot, conv7_b, fc1_W, fc1_b, fc2_W, fc2_b,
           fc3_W, fc3_b):
    e1s, e1d = _pad_edges(edge_index[0], edge_index[1], N1)
    a2s, a2d = _pad_edges(assignment_index_2[0], assignment_index_2[1], N2)
    a3s, a3d = _pad_edges(assignment_index_3[0], assignment_index_3[1], N3)
    e2s, e2d = _pad_edges(edge_index_2[0], edge_index_2[1], N2)
    e3s, e3d = _pad_edges(edge_index_3[0], edge_index_3[1], N3)

    y1, r1 = _tc_first(x, conv1_Wrel, conv1_Wroot, conv1_b)
    m1 = _sc_segsum(y1.reshape(-1, LN), e1s, e1d, s_slices=2, n_out=N1,
                    n_chunks=1, with_counts=False)
    y2, r2 = _tc_mid(m1, r1, conv2_Wrel, conv2_Wroot, conv2_b)
    m2 = _sc_segsum(y2.reshape(-1, LN), e1s, e1d, s_slices=4, n_out=N1,
                    n_chunks=1, with_counts=False)
    y3, r3 = _tc_mid(m2, r2, conv3_Wrel, conv3_Wroot, conv3_b)
    m3 = _sc_segsum(y3.reshape(-1, LN), e1s, e1d, s_slices=4, n_out=N1,
                    n_chunks=1, with_counts=False)
    h, x1 = _tc_last(m3, r3, batch, emit_h=True)

    hf = h.reshape(-1, LN)
    hp2, cnt2 = _sc_segsum(hf, a2s, a2d, s_slices=4, n_out=N2, n_chunks=4,
                           with_counts=True)
    hp3, cnt3 = _sc_segsum(hf, a3s, a3d, s_slices=4, n_out=N3, n_chunks=2,
                           with_counts=True)

    y4, r4 = _tc_post_pool(hp2, cnt2, iso_type_2, conv4_Wrel, conv4_Wroot,
                           conv4_b)
    m4 = _sc_segsum(y4.reshape(-1, LN), e2s, e2d, s_slices=4, n_out=N2,
                    n_chunks=4, with_counts=False)
    y5, r5 = _tc_mid(m4, r4, conv5_Wrel, conv5_Wroot, conv5_b)
    m5 = _sc_segsum(y5.reshape(-1, LN), e2s, e2d, s_slices=4, n_out=N2,
                    n_chunks=4, with_counts=False)
    x2 = _tc_last(m5, r5, batch_2, emit_h=False)

    y6, r6 = _tc_post_pool(hp3, cnt3, iso_type_3, conv6_Wrel, conv6_Wroot,
                           conv6_b)
    m6 = _sc_segsum(y6.reshape(-1, LN), e3s, e3d, s_slices=4, n_out=N3,
                    n_chunks=2, with_counts=False)
    y7, r7 = _tc_mid(m6, r6, conv7_Wrel, conv7_Wroot, conv7_b)
    m7 = _sc_segsum(y7.reshape(-1, LN), e3s, e3d, s_slices=4, n_out=N3,
                    n_chunks=2, with_counts=False)
    x3 = _tc_last(m7, r7, batch_3, emit_h=False)

    return _tc_head(x1, x2, x3, fc1_W, fc1_b, fc2_W, fc2_b, fc3_W, fc3_b)


# speedup vs baseline: 1.8746x; 1.0036x over previous
import functools

import jax
import jax.numpy as jnp
from jax import lax
from jax.experimental import pallas as pl
from jax.experimental.pallas import tpu as pltpu
from jax.experimental.pallas import tpu_sc as plsc

NC, NS, LN = 2, 16, 16
KB = 2048
TI = KB // 128
WT = 1000

N1, G = 10000, 256
N2, I2 = 160000, 12
N3, I3 = 80000, 20


def _elu(v):
    return jnp.where(v > 0, v, jnp.exp(v) - 1.0)


def _pad_edges(src, dst, n_out):
    e = src.shape[0]
    step = NS * KB
    pe = ((e + step - 1) // step) * step
    if pe != e:
        src = jnp.concatenate([src, jnp.zeros((pe - e,), src.dtype)])
        dst = jnp.concatenate([dst, jnp.full((pe - e,), n_out, dst.dtype)])
    return src, dst


def _sc_segsum(y_flat, src, dst, *, s_slices, n_out, n_chunks, with_counts):
    PE = src.shape[0]
    NB = PE // (NS * KB)
    CH = n_out // n_chunks
    assert CH % WT == 0
    NT = CH // WT
    QT = -(-NT // NS)
    SL = s_slices // NC
    S = s_slices

    outs = [jax.ShapeDtypeStruct((S, n_out, LN), jnp.float32)]
    if with_counts:
        outs.append(jax.ShapeDtypeStruct((n_out, LN), jnp.float32))
    scratch = [
        pltpu.VMEM((KB,), jnp.int32),
        pltpu.VMEM((KB,), jnp.int32),
        pltpu.VMEM((KB,), jnp.int32),
        pltpu.VMEM((KB,), jnp.int32),
        pltpu.VMEM((KB, LN), jnp.float32),
        pltpu.VMEM((WT, LN), jnp.float32),
        pltpu.VMEM((WT, LN), jnp.float32),
        pltpu.VMEM_SHARED((CH + 8, LN), jnp.float32),
        pltpu.SemaphoreType.DMA,
    ]
    mesh = plsc.VectorSubcoreMesh(core_axis_name="c", subcore_axis_name="s",
                                  num_cores=NC, num_subcores=NS)

    def body(y_ref, src_ref, dst_ref, *rest):
        if with_counts:
            out_ref, cnt_ref = rest[0], rest[1]
            sc = rest[2:]
        else:
            out_ref = rest[0]
            cnt_ref = None
            sc = rest[1:]
        src_v, dst_v, gidx, didx, rows, zbuf, wbuf, acc, sem = sc
        cid = lax.axis_index("c")
        sid = lax.axis_index("s")
        ebase = sid * (PE // NS)

        z16 = jnp.zeros((LN,), jnp.float32)

        def zfill(i, carry):
            zbuf[i] = z16
            return carry
        lax.fori_loop(0, WT, zfill, 0)

        def zero_acc():
            for q in range(QT):
                t = sid + q * NS
                @pl.when(t < NT)
                def _():
                    pltpu.sync_copy(zbuf, acc.at[pl.ds(t * WT, WT)])

        def run_edges(r0, s, count):
            def bstep(b, carry):
                eb = ebase + b * KB
                pltpu.sync_copy(dst_ref.at[pl.ds(eb, KB)], dst_v)
                if not count:
                    pltpu.sync_copy(src_ref.at[pl.ds(eb, KB)], src_v)

                def istep(j, c2):
                    dv = dst_v[pl.ds(j * LN, LN)]
                    dl = dv - r0
                    okm = (dl >= 0) & (dl < CH)
                    didx[pl.ds(j * LN, LN)] = jnp.where(okm, dl, CH)
                    if not count:
                        sv = src_v[pl.ds(j * LN, LN)]
                        gidx[pl.ds(j * LN, LN)] = sv * S + s
                    return c2
                lax.fori_loop(0, KB // LN, istep, 0)

                if not count:
                    pltpu.async_copy(y_ref.at[gidx], rows, sem).wait()
                pltpu.sync_copy(rows, acc.at[didx], add=True)
                return carry
            lax.fori_loop(0, NB, bstep, 0)

        def writeback(emit):
            for q in range(QT):
                t = sid + q * NS
                @pl.when(t < NT)
                def _():
                    r = t * WT
                    pltpu.sync_copy(acc.at[pl.ds(r, WT)], wbuf)
                    pltpu.sync_copy(wbuf, emit(r))

        for sl in range(SL):
            s = cid * SL + sl
            for c in range(n_chunks):
                r0 = c * CH
                zero_acc()
                plsc.subcore_barrier()
                run_edges(r0, s, False)
                plsc.subcore_barrier()
                writeback(lambda r, s=s, r0=r0:
                          out_ref.at[s, pl.ds(r0 + r, WT)])
                plsc.subcore_barrier()

        if with_counts:
            o16 = jnp.ones((LN,), jnp.float32)

            def ofill(i, carry):
                rows[i] = o16
                return carry
            lax.fori_loop(0, KB, ofill, 0)
            for q in range(max(1, n_chunks // NC)):
                if n_chunks == 1:
                    r0c = 0
                else:
                    r0c = (q * NC + cid) * CH
                zero_acc()
                plsc.subcore_barrier()
                run_edges(r0c, 0, True)
                plsc.subcore_barrier()
                writeback(lambda r, r0c=r0c: cnt_ref.at[pl.ds(r0c + r, WT)])
                plsc.subcore_barrier()

    f = pl.kernel(body, out_type=tuple(outs) if with_counts else outs[0],
                  mesh=mesh, scratch_types=scratch,
                  compiler_params=pltpu.CompilerParams(
                      use_tc_tiling_on_sc=False))
    return f(y_flat, src, dst)


RB = 1000


def _full(spec_shape):
    return pl.BlockSpec(spec_shape, lambda i: tuple(0 for _ in spec_shape))


def _rows(spec_shape, dim=0):
    def imap(i):
        idx = [0] * len(spec_shape)
        idx[dim] = i
        return tuple(idx)
    return pl.BlockSpec(spec_shape, imap)


def _tc_first(x, wrel, wroot, b):
    n, fin = x.shape
    fout = wrel.shape[1]

    def body(x_ref, wr_ref, wo_ref, b_ref, y_ref, r_ref):
        xb = x_ref[...]
        y_ref[...] = xb @ wr_ref[...]
        r_ref[...] = xb @ wo_ref[...] + b_ref[...]

    return pl.pallas_call(
        body,
        grid=(n // RB,),
        in_specs=[_rows((RB, fin)), _full((fin, fout)), _full((fin, fout)),
                  _full((1, fout))],
        out_specs=[_rows((RB, fout)), _rows((RB, fout))],
        out_shape=[jax.ShapeDtypeStruct((n, fout), jnp.float32)] * 2,
    )(x, wrel, wroot, b.reshape(1, -1))


def _tc_mid(m_sm, r_in, wrel, wroot, b):
    s, n, _ = m_sm.shape
    w_in = s * LN
    fout = wrel.shape[1]

    def body(m_ref, r_ref, wr_ref, wo_ref, b_ref, y_ref, r2_ref):
        m = jnp.concatenate([m_ref[i] for i in range(s)], axis=1)
        h = _elu(m + r_ref[...])
        y_ref[...] = h @ wr_ref[...]
        r2_ref[...] = h @ wo_ref[...] + b_ref[...]

    return pl.pallas_call(
        body,
        grid=(n // RB,),
        in_specs=[_rows((s, RB, LN), dim=1), _rows((RB, w_in)),
                  _full((w_in, fout)), _full((w_in, fout)), _full((1, fout))],
        out_specs=[_rows((RB, fout)), _rows((RB, fout))],
        out_shape=[jax.ShapeDtypeStruct((n, fout), jnp.float32)] * 2,
    )(m_sm, r_in, wrel, wroot, b.reshape(1, -1))


def _tc_last(m_sm, r_in, seg, emit_h):
    s, n, _ = m_sm.shape
    w = s * LN
    ng = n // RB

    def body(m_ref, r_ref, bt_ref, *rest):
        if emit_h:
            h_ref, x_ref, pool_acc, cnt_acc = rest
        else:
            x_ref, pool_acc, cnt_acc = rest
        i = pl.program_id(0)
        m = jnp.concatenate([m_ref[k] for k in range(s)], axis=1)
        h = _elu(m + r_ref[...])
        if emit_h:
            h_ref[...] = h
        oh = (bt_ref[...] ==
              lax.broadcasted_iota(jnp.int32, (RB, G), 1)).astype(jnp.float32)
        pool = lax.dot_general(oh, h, (((0,), (0,)), ((), ())))
        cnt = lax.dot_general(oh, jnp.ones((RB, w), jnp.float32),
                              (((0,), (0,)), ((), ())))

        @pl.when(i == 0)
        def _():
            pool_acc[...] = pool
            cnt_acc[...] = cnt

        @pl.when(i > 0)
        def _():
            pool_acc[...] += pool
            cnt_acc[...] += cnt

        @pl.when(i == ng - 1)
        def _():
            x_ref[...] = pool_acc[...] / jnp.maximum(cnt_acc[...], 1.0)

    out_specs = [_full((G, w))]
    out_shape = [jax.ShapeDtypeStruct((G, w), jnp.float32)]
    if emit_h:
        out_specs = [_rows((RB, w))] + out_specs
        out_shape = [jax.ShapeDtypeStruct((n, w), jnp.float32)] + out_shape
    res = pl.pallas_call(
        body,
        grid=(ng,),
        in_specs=[_rows((s, RB, LN), dim=1), _rows((RB, w)), _rows((RB, 1))],
        out_specs=out_specs,
        out_shape=out_shape,
        scratch_shapes=[pltpu.VMEM((G, w), jnp.float32),
                        pltpu.VMEM((G, w), jnp.float32)],
    )(m_sm, r_in, seg.reshape(-1, 1))
    return res if emit_h else res[0]


def _tc_post_pool(hp_sm, cnt, iso, wrel, wroot, b):
    s, n, _ = hp_sm.shape
    w = s * LN
    fi = iso.shape[1]
    fout = wrel.shape[1]
    wrh, wri = wrel[:w], wrel[w:]
    woh, woi = wroot[:w], wroot[w:]

    def body(hp_ref, cnt_ref, iso_ref, wrh_ref, wri_ref, woh_ref, woi_ref,
             b_ref, y_ref, r_ref):
        hsum = jnp.concatenate([hp_ref[k] for k in range(s)], axis=1)
        c = jnp.maximum(cnt_ref[...][:, 0:1], 1.0)
        hm = hsum / c
        iso_b = iso_ref[...]
        y_ref[...] = hm @ wrh_ref[...] + iso_b @ wri_ref[...]
        r_ref[...] = hm @ woh_ref[...] + iso_b @ woi_ref[...] + b_ref[...]

    return pl.pallas_call(
        body,
        grid=(n // RB,),
        in_specs=[_rows((s, RB, LN), dim=1), _rows((RB, LN)), _rows((RB, fi)),
                  _full((w, fout)), _full((fi, fout)),
                  _full((w, fout)), _full((fi, fout)), _full((1, fout))],
        out_specs=[_rows((RB, fout)), _rows((RB, fout))],
        out_shape=[jax.ShapeDtypeStruct((n, fout), jnp.float32)] * 2,
    )(hp_sm, cnt, iso, wrh, wri, woh, woi, b.reshape(1, -1))


def _tc_head(x1, x2, x3, f1w, f1b, f2w, f2b, f3w, f3b):
    def body(x1_ref, x2_ref, x3_ref, w1_ref, b1_ref, w2_ref, b2_ref,
             w3_ref, b3_ref, o_ref):
        z = jnp.concatenate([x1_ref[...], x2_ref[...], x3_ref[...]], axis=1)
        z = _elu(z @ w1_ref[...] + b1_ref[...])
        z = _elu(z @ w2_ref[...] + b2_ref[...])
        z = z @ w3_ref[...] + b3_ref[...]
        mx = jnp.max(z, axis=1, keepdims=True)
        lse = jnp.log(jnp.sum(jnp.exp(z - mx), axis=1, keepdims=True)) + mx
        o_ref[...] = z - lse

    c = f3w.shape[1]
    return pl.pallas_call(
        body,
        grid=(1,),
        in_specs=[_full((G, 64)), _full((G, 64)), _full((G, 64)),
                  _full((192, 64)), _full((1, 64)),
                  _full((64, 32)), _full((1, 32)),
                  _full((32, c)), _full((1, c))],
        out_specs=_full((G, c)),
        out_shape=jax.ShapeDtypeStruct((G, c), jnp.float32),
    )(x1, x2, x3, f1w, f1b.reshape(1, -1), f2w, f2b.reshape(1, -1),
      f3w, f3b.reshape(1, -1))


def kernel(x, edge_index, batch, assignment_index_2, iso_type_2, edge_index_2,
           batch_2, assignment_index_3, iso_type_3, edge_index_3, batch_3,
           conv1_Wrel, conv1_Wroot, conv1_b, conv2_Wrel, conv2_Wroot, conv2_b,
           conv3_Wrel, conv3_Wroot, conv3_b, conv4_Wrel, conv4_Wroot, conv4_b,
           conv5_Wrel, conv5_Wroot, conv5_b, conv6_Wrel, conv6_Wroot, conv6_b,
           conv7_Wrel, conv7_Wroot, conv7_b, fc1_W, fc1_b, fc2_W, fc2_b,
           fc3_W, fc3_b):
    e1s, e1d = _pad_edges(edge_index[0], edge_index[1], N1)
    a2s, a2d = _pad_edges(assignment_index_2[0], assignment_index_2[1], N2)
    a3s, a3d = _pad_edges(assignment_index_3[0], assignment_index_3[1], N3)
    e2s, e2d = _pad_edges(edge_index_2[0], edge_index_2[1], N2)
    e3s, e3d = _pad_edges(edge_index_3[0], edge_index_3[1], N3)

    y1, r1 = _tc_first(x, conv1_Wrel, conv1_Wroot, conv1_b)
    m1 = _sc_segsum(y1.reshape(-1, LN), e1s, e1d, s_slices=2, n_out=N1,
                    n_chunks=1, with_counts=False)
    y2, r2 = _tc_mid(m1, r1, conv2_Wrel, conv2_Wroot, conv2_b)
    m2 = _sc_segsum(y2.reshape(-1, LN), e1s, e1d, s_slices=4, n_out=N1,
                    n_chunks=1, with_counts=False)
    y3, r3 = _tc_mid(m2, r2, conv3_Wrel, conv3_Wroot, conv3_b)
    m3 = _sc_segsum(y3.reshape(-1, LN), e1s, e1d, s_slices=4, n_out=N1,
                    n_chunks=1, with_counts=False)
    h, x1 = _tc_last(m3, r3, batch, emit_h=True)

    hf = h.reshape(-1, LN)
    hp2, cnt2 = _sc_segsum(hf, a2s, a2d, s_slices=4, n_out=N2, n_chunks=4,
                           with_counts=True)
    hp3, cnt3 = _sc_segsum(hf, a3s, a3d, s_slices=4, n_out=N3, n_chunks=2,
                           with_counts=True)

    y4, r4 = _tc_post_pool(hp2, cnt2, iso_type_2, conv4_Wrel, conv4_Wroot,
                           conv4_b)
    m4 = _sc_segsum(y4.reshape(-1, LN), e2s, e2d, s_slices=4, n_out=N2,
                    n_chunks=4, with_counts=False)
    y5, r5 = _tc_mid(m4, r4, conv5_Wrel, conv5_Wroot, conv5_b)
    m5 = _sc_segsum(y5.reshape(-1, LN), e2s, e2d, s_slices=4, n_out=N2,
                    n_chunks=4, with_counts=False)
    x2 = _tc_last(m5, r5, batch_2, emit_h=False)

    y6, r6 = _tc_post_pool(hp3, cnt3, iso_type_3, conv6_Wrel, conv6_Wroot,
                           conv6_b)
    m6 = _sc_segsum(y6.reshape(-1, LN), e3s, e3d, s_slices=4, n_out=N3,
                    n_chunks=2, with_counts=False)
    y7, r7 = _tc_mid(m6, r6, conv7_Wrel, conv7_Wroot, conv7_b)
    m7 = _sc_segsum(y7.reshape(-1, LN), e3s, e3d, s_slices=4, n_out=N3,
                    n_chunks=2, with_counts=False)
    x3 = _tc_last(m7, r7, batch_3, emit_h=False)

    return _tc_head(x1, x2, x3, fc1_W, fc1_b, fc2_W, fc2_b, fc3_W, fc3_b)


# speedup vs baseline: 4.2312x; 2.2571x over previous
import functools

import jax
import jax.numpy as jnp
from jax import lax
from jax.experimental import pallas as pl
from jax.experimental.pallas import tpu as pltpu
from jax.experimental.pallas import tpu_sc as plsc

NC, NS, LN = 2, 16, 16
KB = 1024
WT = 200

N1, G = 10000, 256
N2, I2 = 160000, 12
N3, I3 = 80000, 20


def _elu(v):
    return jnp.where(v > 0, v, jnp.exp(v) - 1.0)


def _pad_edges(src, dst, n_out):
    e = src.shape[0]
    step = 2 * NS * KB
    pe = ((e + step - 1) // step) * step
    if pe != e:
        src = jnp.concatenate([src, jnp.zeros((pe - e,), src.dtype)])
        dst = jnp.concatenate([dst, jnp.full((pe - e,), n_out, dst.dtype)])
    return src, dst


def _prep_gsrc(src, s_slices):
    sl = jnp.arange(s_slices, dtype=src.dtype)[:, None]
    return src[None, :] * s_slices + sl


def _prep_dstc(dst, n_out, n_chunks):
    ch = n_out // n_chunks
    cc = jnp.arange(n_chunks, dtype=dst.dtype)[:, None] * ch
    dl = dst[None, :] - cc
    return jnp.where((dl >= 0) & (dl < ch), dl, ch)


def _sc_segsum(y_flat, gsrc, dstc, *, s_slices, n_out, n_chunks, with_counts):
    PE = gsrc.shape[1]
    NB = PE // (NS * KB)
    CH = n_out // n_chunks
    assert CH % WT == 0
    NT = CH // WT
    QT = -(-NT // NS)
    SL = s_slices // NC
    S = s_slices

    outs = [jax.ShapeDtypeStruct((S, n_out, LN), jnp.float32)]
    if with_counts:
        outs.append(jax.ShapeDtypeStruct((n_out, LN), jnp.float32))
    scratch = [
        pltpu.VMEM((2, KB), jnp.int32),
        pltpu.VMEM((2, KB), jnp.int32),
        pltpu.VMEM((2, KB, LN), jnp.float32),
        pltpu.VMEM((WT, LN), jnp.float32),
        pltpu.VMEM((WT, LN), jnp.float32),
        pltpu.VMEM_SHARED((CH + 8, LN), jnp.float32),
        pltpu.SemaphoreType.DMA,
        pltpu.SemaphoreType.DMA,
    ]
    mesh = plsc.VectorSubcoreMesh(core_axis_name="c", subcore_axis_name="s",
                                  num_cores=NC, num_subcores=NS)

    def body(y_ref, gsrc_ref, dstc_ref, *rest):
        if with_counts:
            out_ref, cnt_ref = rest[0], rest[1]
            sc = rest[2:]
        else:
            out_ref = rest[0]
            cnt_ref = None
            sc = rest[1:]
        gidx, didx, rows, zbuf, wbuf, acc, sem0, sem1 = sc
        sems = (sem0, sem1)
        cid = lax.axis_index("c")
        sid = lax.axis_index("s")
        ebase = sid * (PE // NS)

        z16 = jnp.zeros((LN,), jnp.float32)

        def zfill(i, carry):
            zbuf[i] = z16
            return carry
        lax.fori_loop(0, WT, zfill, 0)

        def zero_acc():
            for q in range(QT):
                t = sid + q * NS
                @pl.when(t < NT)
                def _():
                    pltpu.sync_copy(zbuf, acc.at[pl.ds(t * WT, WT)])

        def run_edges(cc, s, count):
            if count:
                def cstep(b, carry):
                    eb = ebase + b * KB
                    pltpu.sync_copy(dstc_ref.at[cc, pl.ds(eb, KB)],
                                    didx.at[0])
                    pltpu.sync_copy(rows.at[0], acc.at[didx.at[0]], add=True)
                    return carry
                lax.fori_loop(0, NB, cstep, 0)
                return

            def load_idx(b, k):
                eb = ebase + b * KB
                pltpu.sync_copy(dstc_ref.at[cc, pl.ds(eb, KB)], didx.at[k])
                pltpu.sync_copy(gsrc_ref.at[s, pl.ds(eb, KB)], gidx.at[k])

            def start_gather(k):
                pltpu.async_copy(y_ref.at[gidx.at[k]], rows.at[k], sems[k])

            def wait_gather(k):
                pltpu.make_async_copy(y_ref.at[gidx.at[k]], rows.at[k],
                                      sems[k]).wait()

            def do_scatter(k):
                pltpu.sync_copy(rows.at[k], acc.at[didx.at[k]], add=True)

            load_idx(0, 0)
            start_gather(0)

            def pstep(p, carry):
                b0 = 2 * p
                load_idx(b0 + 1, 1)
                start_gather(1)
                wait_gather(0)
                do_scatter(0)

                @pl.when(b0 + 2 < NB)
                def _():
                    load_idx(b0 + 2, 0)
                    start_gather(0)
                wait_gather(1)
                do_scatter(1)
                return carry
            lax.fori_loop(0, NB // 2, pstep, 0)

        def writeback(emit):
            for q in range(QT):
                t = sid + q * NS
                @pl.when(t < NT)
                def _():
                    r = t * WT
                    pltpu.sync_copy(acc.at[pl.ds(r, WT)], wbuf)
                    pltpu.sync_copy(wbuf, emit(r))

        for sl in range(SL):
            s = cid * SL + sl
            for c in range(n_chunks):
                r0 = c * CH
                zero_acc()
                plsc.subcore_barrier()
                run_edges(c, s, False)
                plsc.subcore_barrier()
                writeback(lambda r, s=s, r0=r0:
                          out_ref.at[s, pl.ds(r0 + r, WT)])
                plsc.subcore_barrier()

        if with_counts:
            o16 = jnp.ones((LN,), jnp.float32)

            def ofill(i, carry):
                rows[0, i] = o16
                return carry
            lax.fori_loop(0, KB, ofill, 0)
            for q in range(max(1, n_chunks // NC)):
                if n_chunks == 1:
                    cc = 0
                else:
                    cc = q * NC + cid
                r0c = cc * CH
                zero_acc()
                plsc.subcore_barrier()
                run_edges(cc, 0, True)
                plsc.subcore_barrier()
                writeback(lambda r, r0c=r0c: cnt_ref.at[pl.ds(r0c + r, WT)])
                plsc.subcore_barrier()

    f = pl.kernel(body, out_type=tuple(outs) if with_counts else outs[0],
                  mesh=mesh, scratch_types=scratch,
                  compiler_params=pltpu.CompilerParams(
                      use_tc_tiling_on_sc=False))
    return f(y_flat, gsrc, dstc)


RB = 1000


def _full(spec_shape):
    return pl.BlockSpec(spec_shape, lambda i: tuple(0 for _ in spec_shape))


def _rows(spec_shape, dim=0):
    def imap(i):
        idx = [0] * len(spec_shape)
        idx[dim] = i
        return tuple(idx)
    return pl.BlockSpec(spec_shape, imap)


def _tc_first(x, wrel, wroot, b):
    n, fin = x.shape
    fout = wrel.shape[1]

    def body(x_ref, wr_ref, wo_ref, b_ref, y_ref, r_ref):
        xb = x_ref[...]
        y_ref[...] = xb @ wr_ref[...]
        r_ref[...] = xb @ wo_ref[...] + b_ref[...]

    return pl.pallas_call(
        body,
        grid=(n // RB,),
        in_specs=[_rows((RB, fin)), _full((fin, fout)), _full((fin, fout)),
                  _full((1, fout))],
        out_specs=[_rows((RB, fout)), _rows((RB, fout))],
        out_shape=[jax.ShapeDtypeStruct((n, fout), jnp.float32)] * 2,
    )(x, wrel, wroot, b.reshape(1, -1))


def _tc_mid(m_sm, r_in, wrel, wroot, b):
    s, n, _ = m_sm.shape
    w_in = s * LN
    fout = wrel.shape[1]

    def body(m_ref, r_ref, wr_ref, wo_ref, b_ref, y_ref, r2_ref):
        m = jnp.concatenate([m_ref[i] for i in range(s)], axis=1)
        h = _elu(m + r_ref[...])
        y_ref[...] = h @ wr_ref[...]
        r2_ref[...] = h @ wo_ref[...] + b_ref[...]

    return pl.pallas_call(
        body,
        grid=(n // RB,),
        in_specs=[_rows((s, RB, LN), dim=1), _rows((RB, w_in)),
                  _full((w_in, fout)), _full((w_in, fout)), _full((1, fout))],
        out_specs=[_rows((RB, fout)), _rows((RB, fout))],
        out_shape=[jax.ShapeDtypeStruct((n, fout), jnp.float32)] * 2,
    )(m_sm, r_in, wrel, wroot, b.reshape(1, -1))


def _tc_last(m_sm, r_in, seg, emit_h):
    s, n, _ = m_sm.shape
    w = s * LN
    ng = n // RB

    def body(m_ref, r_ref, bt_ref, *rest):
        if emit_h:
            h_ref, x_ref, pool_acc, cnt_acc = rest
        else:
            x_ref, pool_acc, cnt_acc = rest
        i = pl.program_id(0)
        m = jnp.concatenate([m_ref[k] for k in range(s)], axis=1)
        h = _elu(m + r_ref[...])
        if emit_h:
            h_ref[...] = h
        oh = (bt_ref[...] ==
              lax.broadcasted_iota(jnp.int32, (RB, G), 1)).astype(jnp.float32)
        pool = lax.dot_general(oh, h, (((0,), (0,)), ((), ())))
        cnt = lax.dot_general(oh, jnp.ones((RB, w), jnp.float32),
                              (((0,), (0,)), ((), ())))

        @pl.when(i == 0)
        def _():
            pool_acc[...] = pool
            cnt_acc[...] = cnt

        @pl.when(i > 0)
        def _():
            pool_acc[...] += pool
            cnt_acc[...] += cnt

        @pl.when(i == ng - 1)
        def _():
            x_ref[...] = pool_acc[...] / jnp.maximum(cnt_acc[...], 1.0)

    out_specs = [_full((G, w))]
    out_shape = [jax.ShapeDtypeStruct((G, w), jnp.float32)]
    if emit_h:
        out_specs = [_rows((RB, w))] + out_specs
        out_shape = [jax.ShapeDtypeStruct((n, w), jnp.float32)] + out_shape
    res = pl.pallas_call(
        body,
        grid=(ng,),
        in_specs=[_rows((s, RB, LN), dim=1), _rows((RB, w)), _rows((RB, 1))],
        out_specs=out_specs,
        out_shape=out_shape,
        scratch_shapes=[pltpu.VMEM((G, w), jnp.float32),
                        pltpu.VMEM((G, w), jnp.float32)],
    )(m_sm, r_in, seg.reshape(-1, 1))
    return res if emit_h else res[0]


def _tc_post_pool(hp_sm, cnt, iso, wrel, wroot, b):
    s, n, _ = hp_sm.shape
    w = s * LN
    fi = iso.shape[1]
    fout = wrel.shape[1]
    wrh, wri = wrel[:w], wrel[w:]
    woh, woi = wroot[:w], wroot[w:]

    def body(hp_ref, cnt_ref, iso_ref, wrh_ref, wri_ref, woh_ref, woi_ref,
             b_ref, y_ref, r_ref):
        hsum = jnp.concatenate([hp_ref[k] for k in range(s)], axis=1)
        c = jnp.maximum(cnt_ref[...][:, 0:1], 1.0)
        hm = hsum / c
        iso_b = iso_ref[...]
        y_ref[...] = hm @ wrh_ref[...] + iso_b @ wri_ref[...]
        r_ref[...] = hm @ woh_ref[...] + iso_b @ woi_ref[...] + b_ref[...]

    return pl.pallas_call(
        body,
        grid=(n // RB,),
        in_specs=[_rows((s, RB, LN), dim=1), _rows((RB, LN)), _rows((RB, fi)),
                  _full((w, fout)), _full((fi, fout)),
                  _full((w, fout)), _full((fi, fout)), _full((1, fout))],
        out_specs=[_rows((RB, fout)), _rows((RB, fout))],
        out_shape=[jax.ShapeDtypeStruct((n, fout), jnp.float32)] * 2,
    )(hp_sm, cnt, iso, wrh, wri, woh, woi, b.reshape(1, -1))


def _tc_head(x1, x2, x3, f1w, f1b, f2w, f2b, f3w, f3b):
    def body(x1_ref, x2_ref, x3_ref, w1_ref, b1_ref, w2_ref, b2_ref,
             w3_ref, b3_ref, o_ref):
        z = jnp.concatenate([x1_ref[...], x2_ref[...], x3_ref[...]], axis=1)
        z = _elu(z @ w1_ref[...] + b1_ref[...])
        z = _elu(z @ w2_ref[...] + b2_ref[...])
        z = z @ w3_ref[...] + b3_ref[...]
        mx = jnp.max(z, axis=1, keepdims=True)
        lse = jnp.log(jnp.sum(jnp.exp(z - mx), axis=1, keepdims=True)) + mx
        o_ref[...] = z - lse

    c = f3w.shape[1]
    return pl.pallas_call(
        body,
        grid=(1,),
        in_specs=[_full((G, 64)), _full((G, 64)), _full((G, 64)),
                  _full((192, 64)), _full((1, 64)),
                  _full((64, 32)), _full((1, 32)),
                  _full((32, c)), _full((1, c))],
        out_specs=_full((G, c)),
        out_shape=jax.ShapeDtypeStruct((G, c), jnp.float32),
    )(x1, x2, x3, f1w, f1b.reshape(1, -1), f2w, f2b.reshape(1, -1),
      f3w, f3b.reshape(1, -1))


def kernel(x, edge_index, batch, assignment_index_2, iso_type_2, edge_index_2,
           batch_2, assignment_index_3, iso_type_3, edge_index_3, batch_3,
           conv1_Wrel, conv1_Wroot, conv1_b, conv2_Wrel, conv2_Wroot, conv2_b,
           conv3_Wrel, conv3_Wroot, conv3_b, conv4_Wrel, conv4_Wroot, conv4_b,
           conv5_Wrel, conv5_Wroot, conv5_b, conv6_Wrel, conv6_Wroot, conv6_b,
           conv7_Wrel, conv7_Wroot, conv7_b, fc1_W, fc1_b, fc2_W, fc2_b,
           fc3_W, fc3_b):
    e1s, e1d = _pad_edges(edge_index[0], edge_index[1], N1)
    a2s, a2d = _pad_edges(assignment_index_2[0], assignment_index_2[1], N2)
    a3s, a3d = _pad_edges(assignment_index_3[0], assignment_index_3[1], N3)
    e2s, e2d = _pad_edges(edge_index_2[0], edge_index_2[1], N2)
    e3s, e3d = _pad_edges(edge_index_3[0], edge_index_3[1], N3)

    g1a = _prep_gsrc(e1s, 2)
    g1b = _prep_gsrc(e1s, 4)
    d1 = _prep_dstc(e1d, N1, 1)
    ga2 = _prep_gsrc(a2s, 4)
    da2 = _prep_dstc(a2d, N2, 2)
    ga3 = _prep_gsrc(a3s, 4)
    da3 = _prep_dstc(a3d, N3, 1)
    g2 = _prep_gsrc(e2s, 4)
    d2 = _prep_dstc(e2d, N2, 2)
    g3 = _prep_gsrc(e3s, 4)
    d3 = _prep_dstc(e3d, N3, 1)

    y1, r1 = _tc_first(x, conv1_Wrel, conv1_Wroot, conv1_b)
    m1 = _sc_segsum(y1.reshape(-1, LN), g1a, d1, s_slices=2, n_out=N1,
                    n_chunks=1, with_counts=False)
    y2, r2 = _tc_mid(m1, r1, conv2_Wrel, conv2_Wroot, conv2_b)
    m2 = _sc_segsum(y2.reshape(-1, LN), g1b, d1, s_slices=4, n_out=N1,
                    n_chunks=1, with_counts=False)
    y3, r3 = _tc_mid(m2, r2, conv3_Wrel, conv3_Wroot, conv3_b)
    m3 = _sc_segsum(y3.reshape(-1, LN), g1b, d1, s_slices=4, n_out=N1,
                    n_chunks=1, with_counts=False)
    h, x1 = _tc_last(m3, r3, batch, emit_h=True)

    hf = h.reshape(-1, LN)
    hp2, cnt2 = _sc_segsum(hf, ga2, da2, s_slices=4, n_out=N2, n_chunks=2,
                           with_counts=True)
    hp3, cnt3 = _sc_segsum(hf, ga3, da3, s_slices=4, n_out=N3, n_chunks=1,
                           with_counts=True)

    y4, r4 = _tc_post_pool(hp2, cnt2, iso_type_2, conv4_Wrel, conv4_Wroot,
                           conv4_b)
    m4 = _sc_segsum(y4.reshape(-1, LN), g2, d2, s_slices=4, n_out=N2,
                    n_chunks=2, with_counts=False)
    y5, r5 = _tc_mid(m4, r4, conv5_Wrel, conv5_Wroot, conv5_b)
    m5 = _sc_segsum(y5.reshape(-1, LN), g2, d2, s_slices=4, n_out=N2,
                    n_chunks=2, with_counts=False)
    x2 = _tc_last(m5, r5, batch_2, emit_h=False)

    y6, r6 = _tc_post_pool(hp3, cnt3, iso_type_3, conv6_Wrel, conv6_Wroot,
                           conv6_b)
    m6 = _sc_segsum(y6.reshape(-1, LN), g3, d3, s_slices=4, n_out=N3,
                    n_chunks=1, with_counts=False)
    y7, r7 = _tc_mid(m6, r6, conv7_Wrel, conv7_Wroot, conv7_b)
    m7 = _sc_segsum(y7.reshape(-1, LN), g3, d3, s_slices=4, n_out=N3,
                    n_chunks=1, with_counts=False)
    x3 = _tc_last(m7, r7, batch_3, emit_h=False)

    return _tc_head(x1, x2, x3, fc1_W, fc1_b, fc2_W, fc2_b, fc3_W, fc3_b)
